# baseline (speedup 1.0000x reference)
import functools

import jax
import jax.numpy as jnp
from jax import lax
from jax.experimental import pallas as pl
from jax.experimental.pallas import tpu as pltpu
from jax.experimental.pallas import tpu_sc as plsc

BATCH = 1024
DIM = 16
VOCAB = 100000


def _sc_gather(table, idx):
    B = idx.shape[0]
    V, D = table.shape
    info = plsc.get_sparse_core_info()
    nw = info.num_cores * info.num_subcores
    b_per_w = B // nw

    mesh = plsc.VectorSubcoreMesh(core_axis_name="c", subcore_axis_name="s")

    @functools.partial(
        pl.kernel,
        mesh=mesh,
        out_type=jax.ShapeDtypeStruct((B, D), jnp.float32),
        scratch_types=[
            pltpu.VMEM((b_per_w,), jnp.int32),
            pltpu.VMEM((b_per_w, D), jnp.float32),
            pltpu.SemaphoreType.DMA,
        ],
        compiler_params=pltpu.CompilerParams(use_tc_tiling_on_sc=False),
    )
    def gather_kernel(table_hbm, idx_hbm, out_hbm, idx_v, rows_v, sem):
        wid = lax.axis_index("s") * info.num_cores + lax.axis_index("c")
        base = wid * b_per_w
        pltpu.sync_copy(idx_hbm.at[pl.ds(base, b_per_w)], idx_v)
        pltpu.async_copy(table_hbm.at[idx_v], rows_v, sem).wait()
        pltpu.sync_copy(rows_v, out_hbm.at[pl.ds(base, b_per_w)])

    return gather_kernel(table, idx)


VTILE = 2048
NOUT = 4
NROW = 2
BH = BATCH // NROW


def _tc_project(x, W, b2d):
    B, D = x.shape
    V = W.shape[0]
    nv = pl.cdiv(V, VTILE)
    wlast = V - (nv - 1) * VTILE
    loff = (nv - 1) * VTILE

    def _proj_body(
        x_ref, w_hbm, b_hbm, out_hbm,
        wbuf, bbuf, obuf, wlbuf, blbuf, olbuf,
        wsem, bsem, osem, wlsem, blsem, olsem,
    ):
        i = pl.program_id(0)
        s2 = lax.rem(i, 2)
        s4 = lax.rem(i, NOUT)

        def start_wide_fetch(step):
            ns = lax.rem(step, 2)
            noff = step * VTILE
            pltpu.make_async_copy(
                w_hbm.at[pl.ds(noff, VTILE), :], wbuf.at[ns], wsem.at[ns]
            ).start()
            pltpu.make_async_copy(
                b_hbm.at[:, pl.ds(noff, VTILE)], bbuf.at[ns], bsem.at[ns]
            ).start()

        @pl.when(i == 0)
        def _prologue():
            start_wide_fetch(i)

        @pl.when(i < nv - 1)
        def _wide_step():
            pltpu.make_async_copy(
                w_hbm.at[pl.ds(0, VTILE), :], wbuf.at[s2], wsem.at[s2]
            ).wait()
            pltpu.make_async_copy(
                b_hbm.at[:, pl.ds(0, VTILE)], bbuf.at[s2], bsem.at[s2]
            ).wait()

            @pl.when(i + 1 < nv - 1)
            def _prefetch_wide():
                start_wide_fetch(i + 1)

            @pl.when(i + 1 == nv - 1)
            def _prefetch_last():
                pltpu.make_async_copy(
                    w_hbm.at[pl.ds(loff, wlast), :], wlbuf, wlsem
                ).start()
                pltpu.make_async_copy(
                    b_hbm.at[:, pl.ds(loff, wlast)], blbuf, blsem
                ).start()

            for k in range(NOUT):

                @pl.when((i >= NOUT) & (s4 == k))
                def _wait_prev_out(k=k):
                    for h in range(NROW):
                        pltpu.make_async_copy(
                            obuf.at[k, pl.ds(h * BH, BH), :],
                            out_hbm.at[pl.ds(h * BH, BH), pl.ds(0, VTILE)],
                            osem.at[k, h],
                        ).wait()

            acc = lax.dot_general(
                x_ref[...], wbuf[s2], (((1,), (1,)), ((), ())),
                preferred_element_type=jnp.float32,
            )
            obuf[s4] = acc + bbuf[s2]
            for k in range(NOUT):

                @pl.when(s4 == k)
                def _store_out(k=k):
                    for h in range(NROW):
                        pltpu.make_async_copy(
                            obuf.at[k, pl.ds(h * BH, BH), :],
                            out_hbm.at[pl.ds(h * BH, BH), pl.ds(i * VTILE, VTILE)],
                            osem.at[k, h],
                        ).start()

        @pl.when(i == nv - 1)
        def _last_step():
            pltpu.make_async_copy(
                w_hbm.at[pl.ds(loff, wlast), :], wlbuf, wlsem
            ).wait()
            pltpu.make_async_copy(
                b_hbm.at[:, pl.ds(loff, wlast)], blbuf, blsem
            ).wait()
            acc = lax.dot_general(
                x_ref[...], wlbuf[...], (((1,), (1,)), ((), ())),
                preferred_element_type=jnp.float32,
            )
            olbuf[...] = acc + blbuf[...]
            pltpu.make_async_copy(
                olbuf, out_hbm.at[:, pl.ds(loff, wlast)], olsem
            ).start()
            for step in range(max(nv - 1 - NOUT, 0), nv - 1):
                for h in range(NROW):
                    pltpu.make_async_copy(
                        obuf.at[step % NOUT, pl.ds(h * BH, BH), :],
                        out_hbm.at[pl.ds(h * BH, BH), pl.ds(0, VTILE)],
                        osem.at[step % NOUT, h],
                    ).wait()
            pltpu.make_async_copy(
                olbuf, out_hbm.at[:, pl.ds(loff, wlast)], olsem
            ).wait()

    return pl.pallas_call(
        _proj_body,
        grid=(nv,),
        in_specs=[
            pl.BlockSpec((B, D), lambda i: (0, 0)),
            pl.BlockSpec(memory_space=pl.ANY),
            pl.BlockSpec(memory_space=pl.ANY),
        ],
        out_specs=pl.BlockSpec(memory_space=pl.ANY),
        out_shape=jax.ShapeDtypeStruct((B, V), jnp.float32),
        scratch_shapes=[
            pltpu.VMEM((2, VTILE, D), jnp.float32),
            pltpu.VMEM((2, 1, VTILE), jnp.float32),
            pltpu.VMEM((NOUT, B, VTILE), jnp.float32),
            pltpu.VMEM((wlast, D), jnp.float32),
            pltpu.VMEM((1, wlast), jnp.float32),
            pltpu.VMEM((B, wlast), jnp.float32),
            pltpu.SemaphoreType.DMA((2,)),
            pltpu.SemaphoreType.DMA((2,)),
            pltpu.SemaphoreType.DMA((NOUT, NROW)),
            pltpu.SemaphoreType.DMA,
            pltpu.SemaphoreType.DMA,
            pltpu.SemaphoreType.DMA,
        ],
    )(x, W, b2d)


@jax.jit
def kernel(input, embed, W, b):
    idx = input.astype(jnp.int32)
    x = _sc_gather(embed, idx)
    return _tc_project(x, W, b.reshape(1, -1))


# speedup vs baseline: 1.0033x; 1.0033x over previous
import functools

import jax
import jax.numpy as jnp
from jax import lax
from jax.experimental import pallas as pl
from jax.experimental.pallas import tpu as pltpu
from jax.experimental.pallas import tpu_sc as plsc

BATCH = 1024
DIM = 16
VOCAB = 100000


def _sc_gather(table, idx):
    B = idx.shape[0]
    V, D = table.shape
    info = plsc.get_sparse_core_info()
    nw = info.num_cores * info.num_subcores
    b_per_w = B // nw

    mesh = plsc.VectorSubcoreMesh(core_axis_name="c", subcore_axis_name="s")

    @functools.partial(
        pl.kernel,
        mesh=mesh,
        out_type=jax.ShapeDtypeStruct((B, D), jnp.float32),
        scratch_types=[
            pltpu.VMEM((b_per_w,), jnp.int32),
            pltpu.VMEM((b_per_w, D), jnp.float32),
            pltpu.SemaphoreType.DMA,
        ],
        compiler_params=pltpu.CompilerParams(use_tc_tiling_on_sc=False),
    )
    def gather_kernel(table_hbm, idx_hbm, out_hbm, idx_v, rows_v, sem):
        wid = lax.axis_index("s") * info.num_cores + lax.axis_index("c")
        base = wid * b_per_w
        pltpu.sync_copy(idx_hbm.at[pl.ds(base, b_per_w)], idx_v)
        pltpu.async_copy(table_hbm.at[idx_v], rows_v, sem).wait()
        pltpu.sync_copy(rows_v, out_hbm.at[pl.ds(base, b_per_w)])

    return gather_kernel(table, idx)


VTILE = 2048
NOUT = 4
NROW = 2
BH = BATCH // NROW


def _tc_project(x, W, b2d):
    B, D = x.shape
    V = W.shape[0]
    nv = pl.cdiv(V, VTILE)
    wlast = V - (nv - 1) * VTILE
    loff = (nv - 1) * VTILE

    def _proj_body(
        x_ref, w_hbm, b_hbm, out_hbm,
        wbuf, bbuf, obuf, wlbuf, blbuf, olbuf,
        wsem, bsem, osem, wlsem, blsem, olsem,
    ):
        i = pl.program_id(0)
        s2 = lax.rem(i, 2)
        s4 = lax.rem(i, NOUT)

        def start_wide_fetch(step):
            ns = lax.rem(step, 2)
            noff = step * VTILE
            pltpu.make_async_copy(
                w_hbm.at[pl.ds(noff, VTILE), :], wbuf.at[ns], wsem.at[ns]
            ).start()
            pltpu.make_async_copy(
                b_hbm.at[:, pl.ds(noff, VTILE)], bbuf.at[ns], bsem.at[ns]
            ).start()

        @pl.when(i == 0)
        def _prologue():
            start_wide_fetch(i)

        @pl.when(i < nv - 1)
        def _wide_step():
            pltpu.make_async_copy(
                w_hbm.at[pl.ds(0, VTILE), :], wbuf.at[s2], wsem.at[s2]
            ).wait()
            pltpu.make_async_copy(
                b_hbm.at[:, pl.ds(0, VTILE)], bbuf.at[s2], bsem.at[s2]
            ).wait()

            @pl.when(i + 1 < nv - 1)
            def _prefetch_wide():
                start_wide_fetch(i + 1)

            @pl.when(i + 1 == nv - 1)
            def _prefetch_last():
                pltpu.make_async_copy(
                    w_hbm.at[pl.ds(loff, wlast), :], wlbuf, wlsem
                ).start()
                pltpu.make_async_copy(
                    b_hbm.at[:, pl.ds(loff, wlast)], blbuf, blsem
                ).start()

            for k in range(NOUT):

                @pl.when((i >= NOUT) & (s4 == k))
                def _wait_prev_out(k=k):
                    for h in range(NROW):
                        pltpu.make_async_copy(
                            obuf.at[k, pl.ds(h * BH, BH), :],
                            out_hbm.at[pl.ds(h * BH, BH), pl.ds(0, VTILE)],
                            osem.at[k, h],
                        ).wait()

            acc = lax.dot_general(
                x_ref[...], wbuf[s2], (((1,), (1,)), ((), ())),
                preferred_element_type=jnp.float32,
            )
            obuf[s4] = acc + bbuf[s2]
            for k in range(NOUT):

                @pl.when(s4 == k)
                def _store_out(k=k):
                    for h in range(NROW):
                        pltpu.make_async_copy(
                            obuf.at[k, pl.ds(h * BH, BH), :],
                            out_hbm.at[pl.ds(h * BH, BH), pl.ds(i * VTILE, VTILE)],
                            osem.at[k, h],
                        ).start(priority=(k * NROW + h) % 2)

        @pl.when(i == nv - 1)
        def _last_step():
            pltpu.make_async_copy(
                w_hbm.at[pl.ds(loff, wlast), :], wlbuf, wlsem
            ).wait()
            pltpu.make_async_copy(
                b_hbm.at[:, pl.ds(loff, wlast)], blbuf, blsem
            ).wait()
            acc = lax.dot_general(
                x_ref[...], wlbuf[...], (((1,), (1,)), ((), ())),
                preferred_element_type=jnp.float32,
            )
            olbuf[...] = acc + blbuf[...]
            pltpu.make_async_copy(
                olbuf, out_hbm.at[:, pl.ds(loff, wlast)], olsem
            ).start()
            for step in range(max(nv - 1 - NOUT, 0), nv - 1):
                for h in range(NROW):
                    pltpu.make_async_copy(
                        obuf.at[step % NOUT, pl.ds(h * BH, BH), :],
                        out_hbm.at[pl.ds(h * BH, BH), pl.ds(0, VTILE)],
                        osem.at[step % NOUT, h],
                    ).wait()
            pltpu.make_async_copy(
                olbuf, out_hbm.at[:, pl.ds(loff, wlast)], olsem
            ).wait()

    return pl.pallas_call(
        _proj_body,
        grid=(nv,),
        in_specs=[
            pl.BlockSpec((B, D), lambda i: (0, 0)),
            pl.BlockSpec(memory_space=pl.ANY),
            pl.BlockSpec(memory_space=pl.ANY),
        ],
        out_specs=pl.BlockSpec(memory_space=pl.ANY),
        out_shape=jax.ShapeDtypeStruct((B, V), jnp.float32),
        scratch_shapes=[
            pltpu.VMEM((2, VTILE, D), jnp.float32),
            pltpu.VMEM((2, 1, VTILE), jnp.float32),
            pltpu.VMEM((NOUT, B, VTILE), jnp.float32),
            pltpu.VMEM((wlast, D), jnp.float32),
            pltpu.VMEM((1, wlast), jnp.float32),
            pltpu.VMEM((B, wlast), jnp.float32),
            pltpu.SemaphoreType.DMA((2,)),
            pltpu.SemaphoreType.DMA((2,)),
            pltpu.SemaphoreType.DMA((NOUT, NROW)),
            pltpu.SemaphoreType.DMA,
            pltpu.SemaphoreType.DMA,
            pltpu.SemaphoreType.DMA,
        ],
    )(x, W, b2d)


@jax.jit
def kernel(input, embed, W, b):
    idx = input.astype(jnp.int32)
    x = _sc_gather(embed, idx)
    return _tc_project(x, W, b.reshape(1, -1))


# speedup vs baseline: 2.0020x; 1.9953x over previous
import functools

import jax
import jax.numpy as jnp
from jax import lax
from jax.experimental import pallas as pl
from jax.experimental.pallas import tpu as pltpu
from jax.experimental.pallas import tpu_sc as plsc

BATCH = 1024
DIM = 16
VOCAB = 100000


def _sc_gather(table, idx):
    B = idx.shape[0]
    V, D = table.shape
    info = plsc.get_sparse_core_info()
    nw = info.num_cores * info.num_subcores
    b_per_w = B // nw

    mesh = plsc.VectorSubcoreMesh(core_axis_name="c", subcore_axis_name="s")

    @functools.partial(
        pl.kernel,
        mesh=mesh,
        out_type=jax.ShapeDtypeStruct((B, D), jnp.float32),
        scratch_types=[
            pltpu.VMEM((b_per_w,), jnp.int32),
            pltpu.VMEM((b_per_w, D), jnp.float32),
            pltpu.SemaphoreType.DMA,
        ],
        compiler_params=pltpu.CompilerParams(use_tc_tiling_on_sc=False),
    )
    def gather_kernel(table_hbm, idx_hbm, out_hbm, idx_v, rows_v, sem):
        wid = lax.axis_index("s") * info.num_cores + lax.axis_index("c")
        base = wid * b_per_w
        pltpu.sync_copy(idx_hbm.at[pl.ds(base, b_per_w)], idx_v)
        pltpu.async_copy(table_hbm.at[idx_v], rows_v, sem).wait()
        pltpu.sync_copy(rows_v, out_hbm.at[pl.ds(base, b_per_w)])

    return gather_kernel(table, idx)


VTILE = 2048


def _projT_body(x_ref, w_ref, b_ref, out_ref):
    acc = lax.dot_general(
        w_ref[...], x_ref[...], (((1,), (1,)), ((), ())),
        preferred_element_type=jnp.float32,
    )
    out_ref[...] = acc + b_ref[...]


def _tc_project_t(x, W, bcol):
    B, D = x.shape
    V = W.shape[0]
    nv = pl.cdiv(V, VTILE)
    return pl.pallas_call(
        _projT_body,
        grid=(nv,),
        in_specs=[
            pl.BlockSpec((B, D), lambda i: (0, 0)),
            pl.BlockSpec((VTILE, D), lambda i: (i, 0)),
            pl.BlockSpec((VTILE, 1), lambda i: (i, 0)),
        ],
        out_specs=pl.BlockSpec((VTILE, B), lambda i: (i, 0)),
        out_shape=jax.ShapeDtypeStruct((V, B), jnp.float32),
    )(x, W, bcol)


def _tc_project(x, W, b2d):
    B, D = x.shape
    V = W.shape[0]
    nv = pl.cdiv(V, VTILE)
    wlast = V - (nv - 1) * VTILE
    loff = (nv - 1) * VTILE

    def _proj_body(
        x_ref, w_hbm, b_hbm, out_hbm,
        wbuf, bbuf, obuf, wlbuf, blbuf, olbuf,
        wsem, bsem, osem, wlsem, blsem, olsem,
    ):
        i = pl.program_id(0)
        s2 = lax.rem(i, 2)
        s4 = lax.rem(i, NOUT)

        def start_wide_fetch(step):
            ns = lax.rem(step, 2)
            noff = step * VTILE
            pltpu.make_async_copy(
                w_hbm.at[pl.ds(noff, VTILE), :], wbuf.at[ns], wsem.at[ns]
            ).start()
            pltpu.make_async_copy(
                b_hbm.at[:, pl.ds(noff, VTILE)], bbuf.at[ns], bsem.at[ns]
            ).start()

        @pl.when(i == 0)
        def _prologue():
            start_wide_fetch(i)

        @pl.when(i < nv - 1)
        def _wide_step():
            pltpu.make_async_copy(
                w_hbm.at[pl.ds(0, VTILE), :], wbuf.at[s2], wsem.at[s2]
            ).wait()
            pltpu.make_async_copy(
                b_hbm.at[:, pl.ds(0, VTILE)], bbuf.at[s2], bsem.at[s2]
            ).wait()

            @pl.when(i + 1 < nv - 1)
            def _prefetch_wide():
                start_wide_fetch(i + 1)

            @pl.when(i + 1 == nv - 1)
            def _prefetch_last():
                pltpu.make_async_copy(
                    w_hbm.at[pl.ds(loff, wlast), :], wlbuf, wlsem
                ).start()
                pltpu.make_async_copy(
                    b_hbm.at[:, pl.ds(loff, wlast)], blbuf, blsem
                ).start()

            for k in range(NOUT):

                @pl.when((i >= NOUT) & (s4 == k))
                def _wait_prev_out(k=k):
                    for h in range(NROW):
                        pltpu.make_async_copy(
                            obuf.at[k, pl.ds(h * BH, BH), :],
                            out_hbm.at[pl.ds(h * BH, BH), pl.ds(0, VTILE)],
                            osem.at[k, h],
                        ).wait()

            acc = lax.dot_general(
                x_ref[...], wbuf[s2], (((1,), (1,)), ((), ())),
                preferred_element_type=jnp.float32,
            )
            obuf[s4] = acc + bbuf[s2]
            for k in range(NOUT):

                @pl.when(s4 == k)
                def _store_out(k=k):
                    for h in range(NROW):
                        pltpu.make_async_copy(
                            obuf.at[k, pl.ds(h * BH, BH), :],
                            out_hbm.at[pl.ds(h * BH, BH), pl.ds(i * VTILE, VTILE)],
                            osem.at[k, h],
                        ).start()

        @pl.when(i == nv - 1)
        def _last_step():
            pltpu.make_async_copy(
                w_hbm.at[pl.ds(loff, wlast), :], wlbuf, wlsem
            ).wait()
            pltpu.make_async_copy(
                b_hbm.at[:, pl.ds(loff, wlast)], blbuf, blsem
            ).wait()
            acc = lax.dot_general(
                x_ref[...], wlbuf[...], (((1,), (1,)), ((), ())),
                preferred_element_type=jnp.float32,
            )
            olbuf[...] = acc + blbuf[...]
            pltpu.make_async_copy(
                olbuf, out_hbm.at[:, pl.ds(loff, wlast)], olsem
            ).start()
            for step in range(max(nv - 1 - NOUT, 0), nv - 1):
                for h in range(NROW):
                    pltpu.make_async_copy(
                        obuf.at[step % NOUT, pl.ds(h * BH, BH), :],
                        out_hbm.at[pl.ds(h * BH, BH), pl.ds(0, VTILE)],
                        osem.at[step % NOUT, h],
                    ).wait()
            pltpu.make_async_copy(
                olbuf, out_hbm.at[:, pl.ds(loff, wlast)], olsem
            ).wait()

    return pl.pallas_call(
        _proj_body,
        grid=(nv,),
        in_specs=[
            pl.BlockSpec((B, D), lambda i: (0, 0)),
            pl.BlockSpec(memory_space=pl.ANY),
            pl.BlockSpec(memory_space=pl.ANY),
        ],
        out_specs=pl.BlockSpec(memory_space=pl.ANY),
        out_shape=jax.ShapeDtypeStruct((B, V), jnp.float32),
        scratch_shapes=[
            pltpu.VMEM((2, VTILE, D), jnp.float32),
            pltpu.VMEM((2, 1, VTILE), jnp.float32),
            pltpu.VMEM((NOUT, B, VTILE), jnp.float32),
            pltpu.VMEM((wlast, D), jnp.float32),
            pltpu.VMEM((1, wlast), jnp.float32),
            pltpu.VMEM((B, wlast), jnp.float32),
            pltpu.SemaphoreType.DMA((2,)),
            pltpu.SemaphoreType.DMA((2,)),
            pltpu.SemaphoreType.DMA((NOUT, NROW)),
            pltpu.SemaphoreType.DMA,
            pltpu.SemaphoreType.DMA,
            pltpu.SemaphoreType.DMA,
        ],
    )(x, W, b2d)


@jax.jit
def kernel(input, embed, W, b):
    idx = input.astype(jnp.int32)
    x = _sc_gather(embed, idx)
    return _tc_project_t(x, W, b.reshape(-1, 1)).T


# speedup vs baseline: 2.0099x; 1.0039x over previous
import functools

import jax
import jax.numpy as jnp
from jax import lax
from jax.experimental import pallas as pl
from jax.experimental.pallas import tpu as pltpu
from jax.experimental.pallas import tpu_sc as plsc

BATCH = 1024
DIM = 16
VOCAB = 100000


def _sc_gather(table, idx):
    B = idx.shape[0]
    V, D = table.shape
    info = plsc.get_sparse_core_info()
    nw = info.num_cores * info.num_subcores
    b_per_w = B // nw

    mesh = plsc.VectorSubcoreMesh(core_axis_name="c", subcore_axis_name="s")

    @functools.partial(
        pl.kernel,
        mesh=mesh,
        out_type=jax.ShapeDtypeStruct((B, D), jnp.float32),
        scratch_types=[
            pltpu.VMEM((b_per_w,), jnp.int32),
            pltpu.VMEM((b_per_w, D), jnp.float32),
            pltpu.SemaphoreType.DMA,
        ],
        compiler_params=pltpu.CompilerParams(use_tc_tiling_on_sc=False),
    )
    def gather_kernel(table_hbm, idx_hbm, out_hbm, idx_v, rows_v, sem):
        wid = lax.axis_index("s") * info.num_cores + lax.axis_index("c")
        base = wid * b_per_w
        pltpu.sync_copy(idx_hbm.at[pl.ds(base, b_per_w)], idx_v)
        pltpu.async_copy(table_hbm.at[idx_v], rows_v, sem).wait()
        pltpu.sync_copy(rows_v, out_hbm.at[pl.ds(base, b_per_w)])

    return gather_kernel(table, idx)


VTILE = 2048


def _projT_body(x_ref, w_ref, b_ref, out_ref):
    acc = lax.dot_general(
        w_ref[...], x_ref[...], (((1,), (1,)), ((), ())),
        preferred_element_type=jnp.float32,
    )
    out_ref[...] = acc + b_ref[...]


NOUT = 4


def _tc_project_t(x, W, bcol):
    B, D = x.shape
    V = W.shape[0]
    nv = pl.cdiv(V, VTILE)
    wlast = V - (nv - 1) * VTILE
    loff = (nv - 1) * VTILE

    def body(x_ref, w_ref, b_ref, out_hbm, obuf, osem):
        i = pl.program_id(0)
        s = lax.rem(i, NOUT)

        @pl.when(i >= NOUT)
        def _wait_prev():
            pltpu.make_async_copy(
                obuf.at[s], out_hbm.at[pl.ds(0, VTILE), :], osem.at[s]
            ).wait()

        acc = lax.dot_general(
            w_ref[...], x_ref[...], (((1,), (1,)), ((), ())),
            preferred_element_type=jnp.float32,
        )
        obuf[s] = acc + b_ref[...]

        @pl.when(i < nv - 1)
        def _store():
            pltpu.make_async_copy(
                obuf.at[s], out_hbm.at[pl.ds(i * VTILE, VTILE), :], osem.at[s]
            ).start()

        @pl.when(i == nv - 1)
        def _store_last_and_drain():
            pltpu.make_async_copy(
                obuf.at[s, pl.ds(0, wlast), :],
                out_hbm.at[pl.ds(loff, wlast), :],
                osem.at[s],
            ).start()
            for step in range(max(nv - NOUT, 0), nv):
                k = step % NOUT
                w = VTILE if step < nv - 1 else wlast
                pltpu.make_async_copy(
                    obuf.at[k, pl.ds(0, w), :],
                    out_hbm.at[pl.ds(0, w), :],
                    osem.at[k],
                ).wait()

    return pl.pallas_call(
        body,
        grid=(nv,),
        in_specs=[
            pl.BlockSpec((B, D), lambda i: (0, 0)),
            pl.BlockSpec((VTILE, D), lambda i: (i, 0)),
            pl.BlockSpec((VTILE, 1), lambda i: (i, 0)),
        ],
        out_specs=pl.BlockSpec(memory_space=pl.ANY),
        out_shape=jax.ShapeDtypeStruct((V, B), jnp.float32),
        scratch_shapes=[
            pltpu.VMEM((NOUT, VTILE, B), jnp.float32),
            pltpu.SemaphoreType.DMA((NOUT,)),
        ],
    )(x, W, bcol)


def _tc_project(x, W, b2d):
    B, D = x.shape
    V = W.shape[0]
    nv = pl.cdiv(V, VTILE)
    wlast = V - (nv - 1) * VTILE
    loff = (nv - 1) * VTILE

    def _proj_body(
        x_ref, w_hbm, b_hbm, out_hbm,
        wbuf, bbuf, obuf, wlbuf, blbuf, olbuf,
        wsem, bsem, osem, wlsem, blsem, olsem,
    ):
        i = pl.program_id(0)
        s2 = lax.rem(i, 2)
        s4 = lax.rem(i, NOUT)

        def start_wide_fetch(step):
            ns = lax.rem(step, 2)
            noff = step * VTILE
            pltpu.make_async_copy(
                w_hbm.at[pl.ds(noff, VTILE), :], wbuf.at[ns], wsem.at[ns]
            ).start()
            pltpu.make_async_copy(
                b_hbm.at[:, pl.ds(noff, VTILE)], bbuf.at[ns], bsem.at[ns]
            ).start()

        @pl.when(i == 0)
        def _prologue():
            start_wide_fetch(i)

        @pl.when(i < nv - 1)
        def _wide_step():
            pltpu.make_async_copy(
                w_hbm.at[pl.ds(0, VTILE), :], wbuf.at[s2], wsem.at[s2]
            ).wait()
            pltpu.make_async_copy(
                b_hbm.at[:, pl.ds(0, VTILE)], bbuf.at[s2], bsem.at[s2]
            ).wait()

            @pl.when(i + 1 < nv - 1)
            def _prefetch_wide():
                start_wide_fetch(i + 1)

            @pl.when(i + 1 == nv - 1)
            def _prefetch_last():
                pltpu.make_async_copy(
                    w_hbm.at[pl.ds(loff, wlast), :], wlbuf, wlsem
                ).start()
                pltpu.make_async_copy(
                    b_hbm.at[:, pl.ds(loff, wlast)], blbuf, blsem
                ).start()

            for k in range(NOUT):

                @pl.when((i >= NOUT) & (s4 == k))
                def _wait_prev_out(k=k):
                    for h in range(NROW):
                        pltpu.make_async_copy(
                            obuf.at[k, pl.ds(h * BH, BH), :],
                            out_hbm.at[pl.ds(h * BH, BH), pl.ds(0, VTILE)],
                            osem.at[k, h],
                        ).wait()

            acc = lax.dot_general(
                x_ref[...], wbuf[s2], (((1,), (1,)), ((), ())),
                preferred_element_type=jnp.float32,
            )
            obuf[s4] = acc + bbuf[s2]
            for k in range(NOUT):

                @pl.when(s4 == k)
                def _store_out(k=k):
                    for h in range(NROW):
                        pltpu.make_async_copy(
                            obuf.at[k, pl.ds(h * BH, BH), :],
                            out_hbm.at[pl.ds(h * BH, BH), pl.ds(i * VTILE, VTILE)],
                            osem.at[k, h],
                        ).start()

        @pl.when(i == nv - 1)
        def _last_step():
            pltpu.make_async_copy(
                w_hbm.at[pl.ds(loff, wlast), :], wlbuf, wlsem
            ).wait()
            pltpu.make_async_copy(
                b_hbm.at[:, pl.ds(loff, wlast)], blbuf, blsem
            ).wait()
            acc = lax.dot_general(
                x_ref[...], wlbuf[...], (((1,), (1,)), ((), ())),
                preferred_element_type=jnp.float32,
            )
            olbuf[...] = acc + blbuf[...]
            pltpu.make_async_copy(
                olbuf, out_hbm.at[:, pl.ds(loff, wlast)], olsem
            ).start()
            for step in range(max(nv - 1 - NOUT, 0), nv - 1):
                for h in range(NROW):
                    pltpu.make_async_copy(
                        obuf.at[step % NOUT, pl.ds(h * BH, BH), :],
                        out_hbm.at[pl.ds(h * BH, BH), pl.ds(0, VTILE)],
                        osem.at[step % NOUT, h],
                    ).wait()
            pltpu.make_async_copy(
                olbuf, out_hbm.at[:, pl.ds(loff, wlast)], olsem
            ).wait()

    return pl.pallas_call(
        _proj_body,
        grid=(nv,),
        in_specs=[
            pl.BlockSpec((B, D), lambda i: (0, 0)),
            pl.BlockSpec(memory_space=pl.ANY),
            pl.BlockSpec(memory_space=pl.ANY),
        ],
        out_specs=pl.BlockSpec(memory_space=pl.ANY),
        out_shape=jax.ShapeDtypeStruct((B, V), jnp.float32),
        scratch_shapes=[
            pltpu.VMEM((2, VTILE, D), jnp.float32),
            pltpu.VMEM((2, 1, VTILE), jnp.float32),
            pltpu.VMEM((NOUT, B, VTILE), jnp.float32),
            pltpu.VMEM((wlast, D), jnp.float32),
            pltpu.VMEM((1, wlast), jnp.float32),
            pltpu.VMEM((B, wlast), jnp.float32),
            pltpu.SemaphoreType.DMA((2,)),
            pltpu.SemaphoreType.DMA((2,)),
            pltpu.SemaphoreType.DMA((NOUT, NROW)),
            pltpu.SemaphoreType.DMA,
            pltpu.SemaphoreType.DMA,
            pltpu.SemaphoreType.DMA,
        ],
    )(x, W, b2d)


@jax.jit
def kernel(input, embed, W, b):
    idx = input.astype(jnp.int32)
    x = _sc_gather(embed, idx)
    return _tc_project_t(x, W, b.reshape(-1, 1)).T


# speedup vs baseline: 2.3428x; 1.1656x over previous
import functools

import jax
import jax.numpy as jnp
from jax import lax
from jax.experimental import pallas as pl
from jax.experimental.pallas import tpu as pltpu
from jax.experimental.pallas import tpu_sc as plsc

BATCH = 1024
DIM = 16
VOCAB = 100000


def _sc_gather(table, idx):
    B = idx.shape[0]
    V, D = table.shape
    info = plsc.get_sparse_core_info()
    nw = info.num_cores * info.num_subcores
    b_per_w = B // nw

    mesh = plsc.VectorSubcoreMesh(core_axis_name="c", subcore_axis_name="s")

    @functools.partial(
        pl.kernel,
        mesh=mesh,
        out_type=jax.ShapeDtypeStruct((B, D), jnp.float32),
        scratch_types=[
            pltpu.VMEM((b_per_w,), jnp.int32),
            pltpu.VMEM((b_per_w, D), jnp.float32),
            pltpu.SemaphoreType.DMA,
        ],
        compiler_params=pltpu.CompilerParams(use_tc_tiling_on_sc=False),
    )
    def gather_kernel(table_hbm, idx_hbm, out_hbm, idx_v, rows_v, sem):
        wid = lax.axis_index("s") * info.num_cores + lax.axis_index("c")
        base = wid * b_per_w
        pltpu.sync_copy(idx_hbm.at[pl.ds(base, b_per_w)], idx_v)
        pltpu.async_copy(table_hbm.at[idx_v], rows_v, sem).wait()
        pltpu.sync_copy(rows_v, out_hbm.at[pl.ds(base, b_per_w)])

    return gather_kernel(table, idx)


VTILE = 2048


def _projT_body(x_ref, w_ref, b_ref, out_ref):
    acc = lax.dot_general(
        w_ref[...], x_ref[...], (((1,), (1,)), ((), ())),
        preferred_element_type=jnp.float32,
    )
    out_ref[...] = acc + b_ref[...]


NOUT = 4


def _tc_project_t(x, Wt, bcol):
    B, D = x.shape
    V = Wt.shape[1]
    nv = pl.cdiv(V, VTILE)
    wlast = V - (nv - 1) * VTILE
    loff = (nv - 1) * VTILE

    def body(x_ref, w_ref, b_ref, out_hbm, obuf, osem):
        i = pl.program_id(0)
        s = lax.rem(i, NOUT)

        @pl.when(i >= NOUT)
        def _wait_prev():
            pltpu.make_async_copy(
                obuf.at[s], out_hbm.at[pl.ds(0, VTILE), :], osem.at[s]
            ).wait()

        acc = lax.dot_general(
            w_ref[...], x_ref[...], (((0,), (1,)), ((), ())),
            preferred_element_type=jnp.float32,
        )
        obuf[s] = acc + b_ref[...]

        @pl.when(i < nv - 1)
        def _store():
            pltpu.make_async_copy(
                obuf.at[s], out_hbm.at[pl.ds(i * VTILE, VTILE), :], osem.at[s]
            ).start()

        @pl.when(i == nv - 1)
        def _store_last_and_drain():
            pltpu.make_async_copy(
                obuf.at[s, pl.ds(0, wlast), :],
                out_hbm.at[pl.ds(loff, wlast), :],
                osem.at[s],
            ).start()
            for step in range(max(nv - NOUT, 0), nv):
                k = step % NOUT
                w = VTILE if step < nv - 1 else wlast
                pltpu.make_async_copy(
                    obuf.at[k, pl.ds(0, w), :],
                    out_hbm.at[pl.ds(0, w), :],
                    osem.at[k],
                ).wait()

    return pl.pallas_call(
        body,
        grid=(nv,),
        in_specs=[
            pl.BlockSpec((B, D), lambda i: (0, 0)),
            pl.BlockSpec((D, VTILE), lambda i: (0, i)),
            pl.BlockSpec((VTILE, 1), lambda i: (i, 0)),
        ],
        out_specs=pl.BlockSpec(memory_space=pl.ANY),
        out_shape=jax.ShapeDtypeStruct((V, B), jnp.float32),
        scratch_shapes=[
            pltpu.VMEM((NOUT, VTILE, B), jnp.float32),
            pltpu.SemaphoreType.DMA((NOUT,)),
        ],
    )(x, Wt, bcol)


def _tc_project(x, W, b2d):
    B, D = x.shape
    V = W.shape[0]
    nv = pl.cdiv(V, VTILE)
    wlast = V - (nv - 1) * VTILE
    loff = (nv - 1) * VTILE

    def _proj_body(
        x_ref, w_hbm, b_hbm, out_hbm,
        wbuf, bbuf, obuf, wlbuf, blbuf, olbuf,
        wsem, bsem, osem, wlsem, blsem, olsem,
    ):
        i = pl.program_id(0)
        s2 = lax.rem(i, 2)
        s4 = lax.rem(i, NOUT)

        def start_wide_fetch(step):
            ns = lax.rem(step, 2)
            noff = step * VTILE
            pltpu.make_async_copy(
                w_hbm.at[pl.ds(noff, VTILE), :], wbuf.at[ns], wsem.at[ns]
            ).start()
            pltpu.make_async_copy(
                b_hbm.at[:, pl.ds(noff, VTILE)], bbuf.at[ns], bsem.at[ns]
            ).start()

        @pl.when(i == 0)
        def _prologue():
            start_wide_fetch(i)

        @pl.when(i < nv - 1)
        def _wide_step():
            pltpu.make_async_copy(
                w_hbm.at[pl.ds(0, VTILE), :], wbuf.at[s2], wsem.at[s2]
            ).wait()
            pltpu.make_async_copy(
                b_hbm.at[:, pl.ds(0, VTILE)], bbuf.at[s2], bsem.at[s2]
            ).wait()

            @pl.when(i + 1 < nv - 1)
            def _prefetch_wide():
                start_wide_fetch(i + 1)

            @pl.when(i + 1 == nv - 1)
            def _prefetch_last():
                pltpu.make_async_copy(
                    w_hbm.at[pl.ds(loff, wlast), :], wlbuf, wlsem
                ).start()
                pltpu.make_async_copy(
                    b_hbm.at[:, pl.ds(loff, wlast)], blbuf, blsem
                ).start()

            for k in range(NOUT):

                @pl.when((i >= NOUT) & (s4 == k))
                def _wait_prev_out(k=k):
                    for h in range(NROW):
                        pltpu.make_async_copy(
                            obuf.at[k, pl.ds(h * BH, BH), :],
                            out_hbm.at[pl.ds(h * BH, BH), pl.ds(0, VTILE)],
                            osem.at[k, h],
                        ).wait()

            acc = lax.dot_general(
                x_ref[...], wbuf[s2], (((1,), (1,)), ((), ())),
                preferred_element_type=jnp.float32,
            )
            obuf[s4] = acc + bbuf[s2]
            for k in range(NOUT):

                @pl.when(s4 == k)
                def _store_out(k=k):
                    for h in range(NROW):
                        pltpu.make_async_copy(
                            obuf.at[k, pl.ds(h * BH, BH), :],
                            out_hbm.at[pl.ds(h * BH, BH), pl.ds(i * VTILE, VTILE)],
                            osem.at[k, h],
                        ).start()

        @pl.when(i == nv - 1)
        def _last_step():
            pltpu.make_async_copy(
                w_hbm.at[pl.ds(loff, wlast), :], wlbuf, wlsem
            ).wait()
            pltpu.make_async_copy(
                b_hbm.at[:, pl.ds(loff, wlast)], blbuf, blsem
            ).wait()
            acc = lax.dot_general(
                x_ref[...], wlbuf[...], (((1,), (1,)), ((), ())),
                preferred_element_type=jnp.float32,
            )
            olbuf[...] = acc + blbuf[...]
            pltpu.make_async_copy(
                olbuf, out_hbm.at[:, pl.ds(loff, wlast)], olsem
            ).start()
            for step in range(max(nv - 1 - NOUT, 0), nv - 1):
                for h in range(NROW):
                    pltpu.make_async_copy(
                        obuf.at[step % NOUT, pl.ds(h * BH, BH), :],
                        out_hbm.at[pl.ds(h * BH, BH), pl.ds(0, VTILE)],
                        osem.at[step % NOUT, h],
                    ).wait()
            pltpu.make_async_copy(
                olbuf, out_hbm.at[:, pl.ds(loff, wlast)], olsem
            ).wait()

    return pl.pallas_call(
        _proj_body,
        grid=(nv,),
        in_specs=[
            pl.BlockSpec((B, D), lambda i: (0, 0)),
            pl.BlockSpec(memory_space=pl.ANY),
            pl.BlockSpec(memory_space=pl.ANY),
        ],
        out_specs=pl.BlockSpec(memory_space=pl.ANY),
        out_shape=jax.ShapeDtypeStruct((B, V), jnp.float32),
        scratch_shapes=[
            pltpu.VMEM((2, VTILE, D), jnp.float32),
            pltpu.VMEM((2, 1, VTILE), jnp.float32),
            pltpu.VMEM((NOUT, B, VTILE), jnp.float32),
            pltpu.VMEM((wlast, D), jnp.float32),
            pltpu.VMEM((1, wlast), jnp.float32),
            pltpu.VMEM((B, wlast), jnp.float32),
            pltpu.SemaphoreType.DMA((2,)),
            pltpu.SemaphoreType.DMA((2,)),
            pltpu.SemaphoreType.DMA((NOUT, NROW)),
            pltpu.SemaphoreType.DMA,
            pltpu.SemaphoreType.DMA,
            pltpu.SemaphoreType.DMA,
        ],
    )(x, W, b2d)


@jax.jit
def kernel(input, embed, W, b):
    idx = input.astype(jnp.int32)
    x = _sc_gather(embed, idx)
    return _tc_project_t(x, W.T, b.reshape(-1, 1)).T


# speedup vs baseline: 2.3675x; 1.0106x over previous
import functools

import jax
import jax.numpy as jnp
from jax import lax
from jax.experimental import pallas as pl
from jax.experimental.pallas import tpu as pltpu
from jax.experimental.pallas import tpu_sc as plsc

BATCH = 1024
DIM = 16
VOCAB = 100000


def _sc_gather(table, idx):
    B = idx.shape[0]
    V, D = table.shape
    info = plsc.get_sparse_core_info()
    nw = info.num_cores * info.num_subcores
    b_per_w = B // nw

    mesh = plsc.VectorSubcoreMesh(core_axis_name="c", subcore_axis_name="s")

    @functools.partial(
        pl.kernel,
        mesh=mesh,
        out_type=jax.ShapeDtypeStruct((B, D), jnp.float32),
        scratch_types=[
            pltpu.VMEM((b_per_w,), jnp.int32),
            pltpu.VMEM((b_per_w, D), jnp.float32),
            pltpu.SemaphoreType.DMA,
        ],
        compiler_params=pltpu.CompilerParams(use_tc_tiling_on_sc=False),
    )
    def gather_kernel(table_hbm, idx_hbm, out_hbm, idx_v, rows_v, sem):
        wid = lax.axis_index("s") * info.num_cores + lax.axis_index("c")
        base = wid * b_per_w
        pltpu.sync_copy(idx_hbm.at[pl.ds(base, b_per_w)], idx_v)
        pltpu.async_copy(table_hbm.at[idx_v], rows_v, sem).wait()
        pltpu.sync_copy(rows_v, out_hbm.at[pl.ds(base, b_per_w)])

    return gather_kernel(table, idx)


VTILE = 4096


NOUT = 2


def _tc_project_t(x, Wt, bcol):
    B, D = x.shape
    V = Wt.shape[1]
    nv = pl.cdiv(V, VTILE)
    wlast = V - (nv - 1) * VTILE
    loff = (nv - 1) * VTILE

    def body(x_ref, w_ref, b_ref, out_hbm, obuf, osem):
        i = pl.program_id(0)
        s = lax.rem(i, NOUT)

        @pl.when(i >= NOUT)
        def _wait_prev():
            pltpu.make_async_copy(
                obuf.at[s], out_hbm.at[pl.ds(0, VTILE), :], osem.at[s]
            ).wait()

        acc = lax.dot_general(
            w_ref[...], x_ref[...], (((0,), (1,)), ((), ())),
            preferred_element_type=jnp.float32,
        )
        obuf[s] = acc + b_ref[...]

        @pl.when(i < nv - 1)
        def _store():
            pltpu.make_async_copy(
                obuf.at[s], out_hbm.at[pl.ds(i * VTILE, VTILE), :], osem.at[s]
            ).start()

        @pl.when(i == nv - 1)
        def _store_last_and_drain():
            pltpu.make_async_copy(
                obuf.at[s, pl.ds(0, wlast), :],
                out_hbm.at[pl.ds(loff, wlast), :],
                osem.at[s],
            ).start()
            for step in range(max(nv - NOUT, 0), nv):
                k = step % NOUT
                w = VTILE if step < nv - 1 else wlast
                pltpu.make_async_copy(
                    obuf.at[k, pl.ds(0, w), :],
                    out_hbm.at[pl.ds(0, w), :],
                    osem.at[k],
                ).wait()

    return pl.pallas_call(
        body,
        grid=(nv,),
        in_specs=[
            pl.BlockSpec((B, D), lambda i: (0, 0)),
            pl.BlockSpec((D, VTILE), lambda i: (0, i)),
            pl.BlockSpec((VTILE, 1), lambda i: (i, 0)),
        ],
        out_specs=pl.BlockSpec(memory_space=pl.ANY),
        out_shape=jax.ShapeDtypeStruct((V, B), jnp.float32),
        scratch_shapes=[
            pltpu.VMEM((NOUT, VTILE, B), jnp.float32),
            pltpu.SemaphoreType.DMA((NOUT,)),
        ],
    )(x, Wt, bcol)


@jax.jit
def kernel(input, embed, W, b):
    idx = input.astype(jnp.int32)
    x = _sc_gather(embed, idx)
    return _tc_project_t(x, W.T, b.reshape(-1, 1)).T


# speedup vs baseline: 2.9886x; 1.2623x over previous
import functools

import jax
import jax.numpy as jnp
from jax import lax
from jax.experimental import pallas as pl
from jax.experimental.pallas import tpu as pltpu
from jax.experimental.pallas import tpu_sc as plsc

BATCH = 1024
DIM = 16
VOCAB = 100000


def _sc_gather(table, idx):
    B = idx.shape[0]
    V, D = table.shape
    info = plsc.get_sparse_core_info()
    nw = info.num_cores * info.num_subcores
    b_per_w = B // nw

    mesh = plsc.VectorSubcoreMesh(core_axis_name="c", subcore_axis_name="s")

    @functools.partial(
        pl.kernel,
        mesh=mesh,
        out_type=jax.ShapeDtypeStruct((B, D), jnp.float32),
        scratch_types=[
            pltpu.VMEM((b_per_w,), jnp.int32),
            pltpu.VMEM((b_per_w, D), jnp.float32),
            pltpu.SemaphoreType.DMA,
        ],
        compiler_params=pltpu.CompilerParams(use_tc_tiling_on_sc=False),
    )
    def gather_kernel(table_hbm, idx_hbm, out_hbm, idx_v, rows_v, sem):
        wid = lax.axis_index("s") * info.num_cores + lax.axis_index("c")
        base = wid * b_per_w
        pltpu.sync_copy(idx_hbm.at[pl.ds(base, b_per_w)], idx_v)
        pltpu.async_copy(table_hbm.at[idx_v], rows_v, sem).wait()
        pltpu.sync_copy(rows_v, out_hbm.at[pl.ds(base, b_per_w)])

    return gather_kernel(table, idx)


VTILE = 4096


NOUT = 2


def _tc_project_t(x, Wt):
    B, D = x.shape
    V = Wt.shape[1]
    nv = pl.cdiv(V, VTILE)
    wlast = V - (nv - 1) * VTILE
    loff = (nv - 1) * VTILE

    def body(x_ref, w_ref, out_hbm, obuf, osem):
        i = pl.program_id(0)
        s = lax.rem(i, NOUT)

        @pl.when(i >= NOUT)
        def _wait_prev():
            pltpu.make_async_copy(
                obuf.at[s], out_hbm.at[pl.ds(0, VTILE), :], osem.at[s]
            ).wait()

        obuf[s] = lax.dot_general(
            w_ref[...], x_ref[...], (((0,), (1,)), ((), ())),
            preferred_element_type=jnp.float32,
        )

        @pl.when(i < nv - 1)
        def _store():
            pltpu.make_async_copy(
                obuf.at[s], out_hbm.at[pl.ds(i * VTILE, VTILE), :], osem.at[s]
            ).start()

        @pl.when(i == nv - 1)
        def _store_last_and_drain():
            pltpu.make_async_copy(
                obuf.at[s, pl.ds(0, wlast), :],
                out_hbm.at[pl.ds(loff, wlast), :],
                osem.at[s],
            ).start()
            for step in range(max(nv - NOUT, 0), nv):
                k = step % NOUT
                w = VTILE if step < nv - 1 else wlast
                pltpu.make_async_copy(
                    obuf.at[k, pl.ds(0, w), :],
                    out_hbm.at[pl.ds(0, w), :],
                    osem.at[k],
                ).wait()

    return pl.pallas_call(
        body,
        grid=(nv,),
        in_specs=[
            pl.BlockSpec((B, D), lambda i: (0, 0)),
            pl.BlockSpec((D, VTILE), lambda i: (0, i)),
        ],
        out_specs=pl.BlockSpec(memory_space=pl.ANY),
        out_shape=jax.ShapeDtypeStruct((V, B), jnp.float32),
        scratch_shapes=[
            pltpu.VMEM((NOUT, VTILE, B), jnp.float32),
            pltpu.SemaphoreType.DMA((NOUT,)),
        ],
    )(x, Wt)


@jax.jit
def kernel(input, embed, W, b):
    idx = input.astype(jnp.int32)
    x = _sc_gather(embed, idx)
    xb = jnp.concatenate(
        [x, jnp.ones((x.shape[0], 1), jnp.float32)], axis=1
    ).astype(jnp.bfloat16)
    wtb = jnp.concatenate([W.T, b[None, :]], axis=0).astype(jnp.bfloat16)
    return _tc_project_t(xb, wtb).T


# speedup vs baseline: 3.0011x; 1.0042x over previous
import functools

import jax
import jax.numpy as jnp
from jax import lax
from jax.experimental import pallas as pl
from jax.experimental.pallas import tpu as pltpu
from jax.experimental.pallas import tpu_sc as plsc

BATCH = 1024
DIM = 16
VOCAB = 100000


def _sc_gather(table, idx):
    B = idx.shape[0]
    V, D = table.shape
    info = plsc.get_sparse_core_info()
    nw = info.num_cores * info.num_subcores
    b_per_w = B // nw

    mesh = plsc.VectorSubcoreMesh(core_axis_name="c", subcore_axis_name="s")

    @functools.partial(
        pl.kernel,
        mesh=mesh,
        out_type=jax.ShapeDtypeStruct((B, D), jnp.float32),
        scratch_types=[
            pltpu.VMEM((b_per_w,), jnp.int32),
            pltpu.VMEM((b_per_w, D), jnp.float32),
            pltpu.SemaphoreType.DMA,
        ],
        compiler_params=pltpu.CompilerParams(use_tc_tiling_on_sc=False),
    )
    def gather_kernel(table_hbm, idx_hbm, out_hbm, idx_v, rows_v, sem):
        wid = lax.axis_index("s") * info.num_cores + lax.axis_index("c")
        base = wid * b_per_w
        pltpu.sync_copy(idx_hbm.at[pl.ds(base, b_per_w)], idx_v)
        pltpu.async_copy(table_hbm.at[idx_v], rows_v, sem).wait()
        pltpu.sync_copy(rows_v, out_hbm.at[pl.ds(base, b_per_w)])

    return gather_kernel(table, idx)


VTILE = 2048


NOUT = 4


def _tc_project_t(x, Wt):
    B, D = x.shape
    V = Wt.shape[1]
    nv = pl.cdiv(V, VTILE)
    wlast = V - (nv - 1) * VTILE
    loff = (nv - 1) * VTILE

    def body(x_ref, w_ref, out_hbm, obuf, osem):
        i = pl.program_id(0)
        s = lax.rem(i, NOUT)

        @pl.when(i >= NOUT)
        def _wait_prev():
            pltpu.make_async_copy(
                obuf.at[s], out_hbm.at[pl.ds(0, VTILE), :], osem.at[s]
            ).wait()

        obuf[s] = lax.dot_general(
            w_ref[...], x_ref[...], (((0,), (1,)), ((), ())),
            preferred_element_type=jnp.float32,
        )

        @pl.when(i < nv - 1)
        def _store():
            pltpu.make_async_copy(
                obuf.at[s], out_hbm.at[pl.ds(i * VTILE, VTILE), :], osem.at[s]
            ).start()

        @pl.when(i == nv - 1)
        def _store_last_and_drain():
            pltpu.make_async_copy(
                obuf.at[s, pl.ds(0, wlast), :],
                out_hbm.at[pl.ds(loff, wlast), :],
                osem.at[s],
            ).start()
            for step in range(max(nv - NOUT, 0), nv):
                k = step % NOUT
                w = VTILE if step < nv - 1 else wlast
                pltpu.make_async_copy(
                    obuf.at[k, pl.ds(0, w), :],
                    out_hbm.at[pl.ds(0, w), :],
                    osem.at[k],
                ).wait()

    return pl.pallas_call(
        body,
        grid=(nv,),
        in_specs=[
            pl.BlockSpec((B, D), lambda i: (0, 0)),
            pl.BlockSpec((D, VTILE), lambda i: (0, i)),
        ],
        out_specs=pl.BlockSpec(memory_space=pl.ANY),
        out_shape=jax.ShapeDtypeStruct((V, B), jnp.float32),
        scratch_shapes=[
            pltpu.VMEM((NOUT, VTILE, B), jnp.float32),
            pltpu.SemaphoreType.DMA((NOUT,)),
        ],
    )(x, Wt)


@jax.jit
def kernel(input, embed, W, b):
    idx = input.astype(jnp.int32)
    x = _sc_gather(embed, idx)
    xb = jnp.concatenate(
        [x, jnp.ones((x.shape[0], 1), jnp.float32)], axis=1
    ).astype(jnp.bfloat16)
    wtb = jnp.concatenate([W.T, b[None, :]], axis=0).astype(jnp.bfloat16)
    return _tc_project_t(xb, wtb).T
